# baseline (speedup 1.0000x reference)
import functools

import jax
import jax.numpy as jnp
from jax import lax
from jax.experimental import pallas as pl
from jax.experimental.pallas import tpu as pltpu
from jax.experimental.pallas import tpu_sc as plsc

BATCH = 16384
EMBED_DIM = 16
N_ROWS = 1000000
TAIL_START = (N_ROWS // 128) * 128
NC = 2
NS = 16
NW = NC * NS
B_PER_W = BATCH // NW
GRP = 8
NPAIR = B_PER_W // (2 * GRP)


def _sc_gather_score(user_ids, item_ids, utT, itT, tail_u, tail_i):
    mesh = plsc.VectorSubcoreMesh(core_axis_name="c", subcore_axis_name="s")

    @functools.partial(
        pl.kernel,
        mesh=mesh,
        out_type=[
            jax.ShapeDtypeStruct((BATCH,), jnp.float32),
            jax.ShapeDtypeStruct((BATCH, 128), jnp.float32),
        ],
        scratch_types=[
            pltpu.VMEM((B_PER_W,), jnp.int32),
            pltpu.VMEM((B_PER_W,), jnp.int32),
            pltpu.VMEM((2, GRP, 16, 128), jnp.float32),
            pltpu.VMEM((2, GRP, 16, 128), jnp.float32),
            pltpu.VMEM((2, GRP, 128), jnp.float32),
            pltpu.VMEM((16, 64), jnp.float32),
            pltpu.VMEM((16, 64), jnp.float32),
            pltpu.VMEM((B_PER_W,), jnp.float32),
            pltpu.SemaphoreType.DMA,
            pltpu.SemaphoreType.DMA,
            pltpu.SemaphoreType.DMA,
        ],
        compiler_params=pltpu.CompilerParams(
            use_tc_tiling_on_sc=True, needs_layout_passes=False),
    )
    def score_kernel(uid_hbm, iid_hbm, utT_hbm, itT_hbm, tailu_hbm, taili_hbm,
                     s1_hbm, ig_hbm,
                     idx_u, idx_i, su, si, igb, tu, ti, sbuf,
                     sem0, sem1, semw):
        wid = lax.axis_index("s") * NC + lax.axis_index("c")
        base = wid * B_PER_W
        pltpu.sync_copy(uid_hbm.at[pl.ds(base, B_PER_W)], idx_u)
        pltpu.sync_copy(iid_hbm.at[pl.ds(base, B_PER_W)], idx_i)
        pltpu.sync_copy(tailu_hbm, tu)
        pltpu.sync_copy(taili_hbm, ti)

        lane_iota = lax.iota(jnp.int32, 16)

        def fire(g, slot, sem):
            half = slot
            pb = (g // 2) * 16
            ids_u = idx_u[pl.ds(pb, 16)]
            ids_i = idx_i[pl.ds(pb, 16)]
            for k in range(GRP):
                u = ids_u[half * GRP + k]
                i = ids_i[half * GRP + k]
                au = pl.multiple_of(
                    jnp.minimum((u >> 7) << 7, TAIL_START - 128), 128)
                ai = pl.multiple_of(
                    jnp.minimum((i >> 7) << 7, TAIL_START - 128), 128)
                pltpu.async_copy(
                    utT_hbm.at[:, pl.ds(au, 128)], su.at[slot, k], sem)
                pltpu.async_copy(
                    itT_hbm.at[:, pl.ds(ai, 128)], si.at[slot, k], sem)

        def drain(slot, sem):
            for k in range(GRP):
                pltpu.make_async_copy(
                    utT_hbm.at[:, pl.ds(0, 128)], su.at[slot, k], sem).wait()
                pltpu.make_async_copy(
                    itT_hbm.at[:, pl.ds(0, 128)], si.at[slot, k], sem).wait()

        def wait_igb_write():
            pltpu.make_async_copy(
                igb.at[0], ig_hbm.at[pl.ds(0, GRP)], semw).wait()

        def extract(buf, lane):
            return plsc.load_gather(
                buf, [lane_iota, jnp.broadcast_to(lane, (16,))])

        def compute(g, slot, acc):
            half = slot
            pb = (g // 2) * 16
            ids_u = idx_u[pl.ds(pb, 16)]
            ids_i = idx_i[pl.ds(pb, 16)]
            for k in range(GRP):
                u = ids_u[half * GRP + k]
                i = ids_i[half * GRP + k]
                uvec = jnp.where(
                    u >= TAIL_START,
                    extract(tu, jnp.clip(u - TAIL_START, 0, 63)),
                    extract(su.at[slot, k], u & 127))
                ivec = jnp.where(
                    i >= TAIL_START,
                    extract(ti, jnp.clip(i - TAIL_START, 0, 63)),
                    extract(si.at[slot, k], i & 127))
                igb[slot, k, pl.ds(0, 16)] = ivec
                s = jnp.sum(uvec * ivec, axis=0)
                acc = jnp.where(lane_iota == (slot * GRP + k), s, acc)
            return acc

        fire(0, 0, sem0)

        def pair_body(p, _):
            g0 = 2 * p
            fire(g0 + 1, 1, sem1)
            drain(0, sem0)
            @pl.when(p >= 1)
            def _():
                wait_igb_write()
                wait_igb_write()
            acc = compute(g0, 0, jnp.zeros((16,), jnp.float32))

            @pl.when(p + 1 < NPAIR)
            def _():
                fire(g0 + 2, 0, sem0)

            pltpu.async_copy(
                igb.at[0], ig_hbm.at[pl.ds(base + g0 * GRP, GRP)], semw)
            drain(1, sem1)
            acc = compute(g0 + 1, 1, acc)
            pltpu.async_copy(
                igb.at[1], ig_hbm.at[pl.ds(base + (g0 + 1) * GRP, GRP)], semw)
            sbuf[pl.ds(p * 16, 16)] = acc
            return ()

        lax.fori_loop(0, NPAIR, pair_body, (), unroll=False)
        wait_igb_write()
        wait_igb_write()
        pltpu.sync_copy(sbuf, s1_hbm.at[pl.ds(base, B_PER_W)])

    return score_kernel(user_ids, item_ids, utT, itT, tail_u, tail_i)


def _tc_combine_body(uf_ref, w1_ref, b1_ref, w2_ref, b2_ref, ig_ref, s1_ref,
                     out_ref):
    h = jnp.maximum(
        jnp.dot(uf_ref[...], w1_ref[...], preferred_element_type=jnp.float32)
        + b1_ref[...], 0.0)
    c = jnp.dot(h, w2_ref[...], preferred_element_type=jnp.float32) \
        + b2_ref[...]
    s2 = jnp.sum(c * ig_ref[:, :EMBED_DIM], axis=1)
    out_ref[...] = s1_ref[...] + s2


def _tc_combine(user_features, W1, b1, W2, b2, ig, s1):
    blk = 2048
    return pl.pallas_call(
        _tc_combine_body,
        grid=(BATCH // blk,),
        in_specs=[
            pl.BlockSpec((blk, 64), lambda i: (i, 0)),
            pl.BlockSpec((64, 32), lambda i: (0, 0)),
            pl.BlockSpec((1, 32), lambda i: (0, 0)),
            pl.BlockSpec((32, EMBED_DIM), lambda i: (0, 0)),
            pl.BlockSpec((1, EMBED_DIM), lambda i: (0, 0)),
            pl.BlockSpec((blk, 128), lambda i: (i, 0)),
            pl.BlockSpec((blk,), lambda i: (i,)),
        ],
        out_specs=pl.BlockSpec((blk,), lambda i: (i,)),
        out_shape=jax.ShapeDtypeStruct((BATCH,), jnp.float32),
    )(user_features, W1, b1.reshape(1, 32), W2, b2.reshape(1, EMBED_DIM),
      ig, s1)


def kernel(user_ids, item_ids, user_features, cf_user_table, cf_item_table,
           W1, b1, W2, b2):
    utT = cf_user_table.T
    itT = cf_item_table.T
    tail_u = lax.slice(utT, (0, TAIL_START), (EMBED_DIM, N_ROWS))
    tail_i = lax.slice(itT, (0, TAIL_START), (EMBED_DIM, N_ROWS))
    s1, ig = _sc_gather_score(user_ids, item_ids, utT, itT, tail_u, tail_i)
    return _tc_combine(user_features, W1, b1, W2, b2, ig, s1)


# speedup vs baseline: 1.1572x; 1.1572x over previous
import functools

import jax
import jax.numpy as jnp
from jax import lax
from jax.experimental import pallas as pl
from jax.experimental.pallas import tpu as pltpu
from jax.experimental.pallas import tpu_sc as plsc

BATCH = 16384
EMBED_DIM = 16
N_ROWS = 1000000
TAIL_START = (N_ROWS // 128) * 128
NC = 2
NS = 16
NW = NC * NS
B_PER_W = BATCH // NW
GRP = 8
NPAIR = B_PER_W // (2 * GRP)


def _sc_gather_score(user_ids, item_ids, utT, itT, tail_u, tail_i):
    mesh = plsc.VectorSubcoreMesh(core_axis_name="c", subcore_axis_name="s")

    @functools.partial(
        pl.kernel,
        mesh=mesh,
        out_type=[
            jax.ShapeDtypeStruct((BATCH,), jnp.float32),
            jax.ShapeDtypeStruct((EMBED_DIM, BATCH), jnp.float32),
        ],
        scratch_types=[
            pltpu.VMEM((B_PER_W,), jnp.int32),
            pltpu.VMEM((B_PER_W,), jnp.int32),
            pltpu.VMEM((2, GRP, 16, 128), jnp.float32),
            pltpu.VMEM((2, GRP, 16, 128), jnp.float32),
            pltpu.VMEM((16, 128), jnp.float32),
            pltpu.VMEM((16, 64), jnp.float32),
            pltpu.VMEM((16, 64), jnp.float32),
            pltpu.VMEM((B_PER_W,), jnp.float32),
            pltpu.SemaphoreType.DMA,
            pltpu.SemaphoreType.DMA,
            pltpu.SemaphoreType.DMA,
        ],
        compiler_params=pltpu.CompilerParams(
            use_tc_tiling_on_sc=True, needs_layout_passes=False),
    )
    def score_kernel(uid_hbm, iid_hbm, utT_hbm, itT_hbm, tailu_hbm, taili_hbm,
                     s1_hbm, ig_hbm,
                     idx_u, idx_i, su, si, igb, tu, ti, sbuf,
                     sem0, sem1, semw):
        wid = lax.axis_index("s") * NC + lax.axis_index("c")
        base = wid * B_PER_W
        pltpu.sync_copy(uid_hbm.at[pl.ds(base, B_PER_W)], idx_u)
        pltpu.sync_copy(iid_hbm.at[pl.ds(base, B_PER_W)], idx_i)
        pltpu.sync_copy(tailu_hbm, tu)
        pltpu.sync_copy(taili_hbm, ti)

        lane_iota = lax.iota(jnp.int32, 16)

        def fire(g, slot, sem):
            half = slot
            pb = (g // 2) * 16
            ids_u = idx_u[pl.ds(pb, 16)]
            ids_i = idx_i[pl.ds(pb, 16)]
            for k in range(GRP):
                u = ids_u[half * GRP + k]
                i = ids_i[half * GRP + k]
                au = pl.multiple_of(
                    jnp.minimum((u >> 7) << 7, TAIL_START - 128), 128)
                ai = pl.multiple_of(
                    jnp.minimum((i >> 7) << 7, TAIL_START - 128), 128)
                pltpu.async_copy(
                    utT_hbm.at[:, pl.ds(au, 128)], su.at[slot, k], sem)
                pltpu.async_copy(
                    itT_hbm.at[:, pl.ds(ai, 128)], si.at[slot, k], sem)

        def drain(slot, sem):
            for k in range(GRP):
                pltpu.make_async_copy(
                    utT_hbm.at[:, pl.ds(0, 128)], su.at[slot, k], sem).wait()
                pltpu.make_async_copy(
                    itT_hbm.at[:, pl.ds(0, 128)], si.at[slot, k], sem).wait()

        def extract(buf, lane):
            return plsc.load_gather(
                buf, [lane_iota, jnp.broadcast_to(lane, (16,))])

        def compute(g, slot, acc):
            half = slot
            pb = (g // 2) * 16
            ids_u = idx_u[pl.ds(pb, 16)]
            ids_i = idx_i[pl.ds(pb, 16)]
            for k in range(GRP):
                u = ids_u[half * GRP + k]
                i = ids_i[half * GRP + k]
                uvec = jnp.where(
                    u >= TAIL_START,
                    extract(tu, jnp.clip(u - TAIL_START, 0, 63)),
                    extract(su.at[slot, k], u & 127))
                ivec = jnp.where(
                    i >= TAIL_START,
                    extract(ti, jnp.clip(i - TAIL_START, 0, 63)),
                    extract(si.at[slot, k], i & 127))
                col = (g & 15) * GRP + k
                plsc.store_scatter(
                    igb, [lane_iota, jnp.broadcast_to(col, (16,))], ivec)
                s = jnp.sum(uvec * ivec, axis=0)
                acc = jnp.where(lane_iota == (slot * GRP + k), s, acc)
            return acc

        fire(0, 0, sem0)

        def pair_body(p, _):
            g0 = 2 * p
            fire(g0 + 1, 1, sem1)
            drain(0, sem0)
            acc = compute(g0, 0, jnp.zeros((16,), jnp.float32))

            @pl.when(p + 1 < NPAIR)
            def _():
                fire(g0 + 2, 0, sem0)

            drain(1, sem1)
            acc = compute(g0 + 1, 1, acc)
            sbuf[pl.ds(p * 16, 16)] = acc

            @pl.when((p & 7) == 7)
            def _():
                tile = pl.multiple_of(((p >> 3) << 7), 128)
                pltpu.sync_copy(
                    igb, ig_hbm.at[:, pl.ds(base + tile, 128)])
            return ()

        lax.fori_loop(0, NPAIR, pair_body, (), unroll=False)
        pltpu.sync_copy(sbuf, s1_hbm.at[pl.ds(base, B_PER_W)])

    return score_kernel(user_ids, item_ids, utT, itT, tail_u, tail_i)


def _tc_combine_body(ufT_ref, w1T_ref, b1_ref, w2T_ref, b2_ref, igT_ref,
                     s1_ref, out_ref):
    hT = jnp.maximum(
        jnp.dot(w1T_ref[...], ufT_ref[...],
                preferred_element_type=jnp.float32) + b1_ref[...], 0.0)
    cT = jnp.dot(w2T_ref[...], hT, preferred_element_type=jnp.float32) \
        + b2_ref[...]
    s2 = jnp.sum(cT * igT_ref[...], axis=0)
    out_ref[...] = s1_ref[...] + s2


def _tc_combine(user_features, W1, b1, W2, b2, igT, s1):
    blk = 2048
    ufT = user_features.T
    w1T = W1.T
    w2T = W2.T
    return pl.pallas_call(
        _tc_combine_body,
        grid=(BATCH // blk,),
        in_specs=[
            pl.BlockSpec((64, blk), lambda i: (0, i)),
            pl.BlockSpec((32, 64), lambda i: (0, 0)),
            pl.BlockSpec((32, 1), lambda i: (0, 0)),
            pl.BlockSpec((EMBED_DIM, 32), lambda i: (0, 0)),
            pl.BlockSpec((EMBED_DIM, 1), lambda i: (0, 0)),
            pl.BlockSpec((EMBED_DIM, blk), lambda i: (0, i)),
            pl.BlockSpec((blk,), lambda i: (i,)),
        ],
        out_specs=pl.BlockSpec((blk,), lambda i: (i,)),
        out_shape=jax.ShapeDtypeStruct((BATCH,), jnp.float32),
    )(ufT, w1T, b1.reshape(32, 1), w2T, b2.reshape(EMBED_DIM, 1), igT, s1)


def kernel(user_ids, item_ids, user_features, cf_user_table, cf_item_table,
           W1, b1, W2, b2):
    utT = cf_user_table.T
    itT = cf_item_table.T
    tail_u = lax.slice(utT, (0, TAIL_START), (EMBED_DIM, N_ROWS))
    tail_i = lax.slice(itT, (0, TAIL_START), (EMBED_DIM, N_ROWS))
    s1, igT = _sc_gather_score(user_ids, item_ids, utT, itT, tail_u, tail_i)
    return _tc_combine(user_features, W1, b1, W2, b2, igT, s1)
